# Initial kernel scaffold; baseline (speedup 1.0000x reference)
import jax
import jax.numpy as jnp
from jax.experimental import pallas as pl


def kernel(x, edge_index_follows, edge_index_likes, W_follows, b_follows, Wq_follows, bq_follows, Wk_follows, bk_follows, Wv_follows, bv_follows, W_likes, b_likes, Wq_likes, bq_likes, Wk_likes, bk_likes, Wv_likes, bv_likes, ln_gamma, ln_beta):
    raise NotImplementedError("write your pallas kernel here")


# speedup vs baseline: 4.6734x; 4.6734x over previous
import functools

import numpy as np
import jax
import jax.numpy as jnp
from jax import lax
from jax.experimental import pallas as pl
from jax.experimental.pallas import tpu as pltpu
from jax.experimental.pallas import tpu_sc as plsc

N = 10000
D = 128
E = 160000
H = 4
DH = D // H

NC = 2
NS = 16
CHUNK = 128
CHUNKS = 79
E_PAD = NS * CHUNKS * CHUNK
N_PAD = 10240
ROWS_PER_SUB = N_PAD // NS

BLK = 2000


def _dense_body(x_ref, *refs):
    wrefs = refs[:16]
    out_refs = refs[16:]
    x = x_ref[...]
    scale = 1.0 / np.sqrt(DH)
    for et in range(2):
        (w, b, wq, bq, wk, bk, wv, bv) = wrefs[8 * et:8 * et + 8]
        o_ref = out_refs[et]
        wh = jnp.dot(x, w[...], preferred_element_type=jnp.float32) + b[...]
        q = jnp.dot(wh, wq[...], preferred_element_type=jnp.float32) + bq[...]
        k = jnp.dot(wh, wk[...], preferred_element_type=jnp.float32) + bk[...]
        v = jnp.dot(wh, wv[...], preferred_element_type=jnp.float32) + bv[...]
        qs = [q[:, h * DH:(h + 1) * DH] for h in range(H)]
        ks = [k[:, h * DH:(h + 1) * DH] for h in range(H)]
        vs = [v[:, h * DH:(h + 1) * DH] for h in range(H)]
        for h in range(H):
            logit = [jnp.sum(qs[h] * ks[g], axis=1, keepdims=True) * scale
                     for g in range(H)]
            m = jnp.maximum(jnp.maximum(logit[0], logit[1]),
                            jnp.maximum(logit[2], logit[3]))
            e = [jnp.exp(l - m) for l in logit]
            s = e[0] + e[1] + e[2] + e[3]
            o = (e[0] * vs[0] + e[1] * vs[1] + e[2] * vs[2] + e[3] * vs[3]) / s
            o_ref[:, h * DH:(h + 1) * DH] = o


def _dense(x, wts):
    nblk = N // BLK
    row_spec = pl.BlockSpec((BLK, D), lambda i: (i, 0))
    mat_spec = pl.BlockSpec((D, D), lambda i: (0, 0))
    vec_spec = pl.BlockSpec((1, D), lambda i: (0, 0))
    in_specs = [row_spec]
    for j in range(16):
        in_specs.append(mat_spec if j % 2 == 0 else vec_spec)
    return pl.pallas_call(
        _dense_body,
        grid=(nblk,),
        in_specs=in_specs,
        out_specs=[row_spec, row_spec],
        out_shape=[jax.ShapeDtypeStruct((N, D), jnp.float32)] * 2,
    )(x, *wts)


@functools.lru_cache(maxsize=1)
def _make_sc_aggregate():
    mesh = plsc.VectorSubcoreMesh(core_axis_name="c", subcore_axis_name="s")
    return pl.kernel(
        _sc_aggregate_body,
        mesh=mesh,
            out_type=[
            jax.ShapeDtypeStruct((N_PAD, D), jnp.float32),
            jax.ShapeDtypeStruct((N_PAD,), jnp.float32),
            jax.ShapeDtypeStruct((N_PAD, D), jnp.float32),
            jax.ShapeDtypeStruct((N_PAD,), jnp.float32),
        ],
        scratch_types=[
            pltpu.VMEM((CHUNKS, CHUNK), jnp.int32),
            pltpu.VMEM((CHUNKS, CHUNK), jnp.int32),
            pltpu.VMEM((CHUNK, D), jnp.float32),
            pltpu.VMEM((CHUNK,), jnp.float32),
            pltpu.VMEM_SHARED((N_PAD, D), jnp.float32),
            pltpu.VMEM_SHARED((N_PAD,), jnp.float32),
            pltpu.SemaphoreType.DMA,
        ],
    )


def _sc_aggregate_body(out_f_hbm, out_l_hbm, src_f_hbm, dst_f_hbm,
                  src_l_hbm, dst_l_hbm, zeros2_hbm, zeros1_hbm, ones_hbm,
                  sum_f_hbm, cnt_f_hbm, sum_l_hbm, cnt_l_hbm,
                  src_v, dst_v, msg_v, ones_v, acc_s, cnt_s, sem):
    cid = lax.axis_index("c")
    sid = lax.axis_index("s")
    row0 = sid * ROWS_PER_SUB
    pltpu.sync_copy(zeros2_hbm, acc_s.at[pl.ds(row0, ROWS_PER_SUB)])
    pltpu.sync_copy(zeros1_hbm, cnt_s.at[pl.ds(row0, ROWS_PER_SUB)])
    pltpu.sync_copy(ones_hbm, ones_v)
    plsc.subcore_barrier()

    def run(table_hbm, src_hbm, dst_hbm):
        pltpu.sync_copy(src_hbm.at[sid], src_v)
        pltpu.sync_copy(dst_hbm.at[sid], dst_v)

        def body(j, carry):
            pltpu.async_copy(table_hbm.at[src_v.at[j]], msg_v, sem).wait()
            pltpu.sync_copy(msg_v, acc_s.at[dst_v.at[j]], add=True)
            pltpu.sync_copy(ones_v, cnt_s.at[dst_v.at[j]], add=True)
            return carry

        lax.fori_loop(0, CHUNKS, body, 0)

    @pl.when(cid == 0)
    def _():
        run(out_f_hbm, src_f_hbm, dst_f_hbm)

    @pl.when(cid == 1)
    def _():
        run(out_l_hbm, src_l_hbm, dst_l_hbm)

    plsc.subcore_barrier()

    @pl.when(cid == 0)
    def _():
        pltpu.sync_copy(acc_s.at[pl.ds(row0, ROWS_PER_SUB)],
                        sum_f_hbm.at[pl.ds(row0, ROWS_PER_SUB)])
        pltpu.sync_copy(cnt_s.at[pl.ds(row0, ROWS_PER_SUB)],
                        cnt_f_hbm.at[pl.ds(row0, ROWS_PER_SUB)])

    @pl.when(cid == 1)
    def _():
        pltpu.sync_copy(acc_s.at[pl.ds(row0, ROWS_PER_SUB)],
                        sum_l_hbm.at[pl.ds(row0, ROWS_PER_SUB)])
        pltpu.sync_copy(cnt_s.at[pl.ds(row0, ROWS_PER_SUB)],
                        cnt_l_hbm.at[pl.ds(row0, ROWS_PER_SUB)])


def _combine_body(sf_ref, cf_ref, sl_ref, cl_ref, x_ref, g_ref, b_ref, o_ref):
    h = (sf_ref[...] / jnp.maximum(cf_ref[...], 1.0)
         + sl_ref[...] / jnp.maximum(cl_ref[...], 1.0)
         + x_ref[...])
    mu = jnp.mean(h, axis=1, keepdims=True)
    d = h - mu
    var = jnp.mean(d * d, axis=1, keepdims=True)
    o_ref[...] = d * lax.rsqrt(var + 1e-5) * g_ref[...] + b_ref[...]


def _combine(sum_f, cnt_f, sum_l, cnt_l, x, gamma, beta):
    nblk = N // BLK
    row_spec = pl.BlockSpec((BLK, D), lambda i: (i, 0))
    col_spec = pl.BlockSpec((BLK, 1), lambda i: (i, 0))
    vec_spec = pl.BlockSpec((1, D), lambda i: (0, 0))
    return pl.pallas_call(
        _combine_body,
        grid=(nblk,),
        in_specs=[row_spec, col_spec, row_spec, col_spec, row_spec,
                  vec_spec, vec_spec],
        out_specs=row_spec,
        out_shape=jax.ShapeDtypeStruct((N, D), jnp.float32),
    )(sum_f, cnt_f, sum_l, cnt_l, x, gamma, beta)


def _prep_edges(ei):
    src = ei[0].astype(jnp.int32)
    dst = ei[1].astype(jnp.int32)
    pad = E_PAD - E
    src = jnp.concatenate([src, jnp.zeros((pad,), jnp.int32)])
    dst = jnp.concatenate([dst, jnp.full((pad,), N, jnp.int32)])
    return src.reshape(NS, CHUNKS, CHUNK), dst.reshape(NS, CHUNKS, CHUNK)


def kernel(x, edge_index_follows, edge_index_likes,
           W_follows, b_follows, Wq_follows, bq_follows, Wk_follows,
           bk_follows, Wv_follows, bv_follows,
           W_likes, b_likes, Wq_likes, bq_likes, Wk_likes, bk_likes,
           Wv_likes, bv_likes, ln_gamma, ln_beta):
    wts = []
    for (w, b, wq, bq, wk, bk, wv, bv) in (
            (W_follows, b_follows, Wq_follows, bq_follows, Wk_follows,
             bk_follows, Wv_follows, bv_follows),
            (W_likes, b_likes, Wq_likes, bq_likes, Wk_likes, bk_likes,
             Wv_likes, bv_likes)):
        wts += [w.T, b.reshape(1, D), wq.T, bq.reshape(1, D),
                wk.T, bk.reshape(1, D), wv.T, bv.reshape(1, D)]
    out_f, out_l = _dense(x, wts)

    src_f, dst_f = _prep_edges(edge_index_follows)
    src_l, dst_l = _prep_edges(edge_index_likes)
    zeros2 = jnp.zeros((ROWS_PER_SUB, D), jnp.float32)
    zeros1 = jnp.zeros((ROWS_PER_SUB,), jnp.float32)
    ones = jnp.ones((CHUNK,), jnp.float32)
    sum_f, cnt_f, sum_l, cnt_l = _make_sc_aggregate()(
        out_f, out_l, src_f, dst_f, src_l, dst_l, zeros2, zeros1, ones)

    return _combine(sum_f[:N], cnt_f[:N].reshape(N, 1),
                    sum_l[:N], cnt_l[:N].reshape(N, 1),
                    x, ln_gamma.reshape(1, D), ln_beta.reshape(1, D))


# speedup vs baseline: 5.1142x; 1.0943x over previous
import functools

import numpy as np
import jax
import jax.numpy as jnp
from jax import lax
from jax.experimental import pallas as pl
from jax.experimental.pallas import tpu as pltpu
from jax.experimental.pallas import tpu_sc as plsc

N = 10000
D = 128
E = 160000
H = 4
DH = D // H

NS = 16
CHUNK = 128
CHUNKS = 79
E_PAD = NS * CHUNKS * CHUNK
N_PAD = 10240
ROWS_PER_SUB = N_PAD // NS

BLK = 2000


def _dense_body(x_ref, *refs):
    wrefs = refs[:16]
    out_refs = refs[16:]
    x = x_ref[...]
    scale = 1.0 / np.sqrt(DH)
    for et in range(2):
        (w, b, wq, bq, wk, bk, wv, bv) = wrefs[8 * et:8 * et + 8]
        o_ref = out_refs[et]
        wh = jnp.dot(x, w[...], preferred_element_type=jnp.float32) + b[...]
        q = jnp.dot(wh, wq[...], preferred_element_type=jnp.float32) + bq[...]
        k = jnp.dot(wh, wk[...], preferred_element_type=jnp.float32) + bk[...]
        v = jnp.dot(wh, wv[...], preferred_element_type=jnp.float32) + bv[...]
        qs = [q[:, h * DH:(h + 1) * DH] for h in range(H)]
        ks = [k[:, h * DH:(h + 1) * DH] for h in range(H)]
        vs = [v[:, h * DH:(h + 1) * DH] for h in range(H)]
        for h in range(H):
            logit = [jnp.sum(qs[h] * ks[g], axis=1, keepdims=True) * scale
                     for g in range(H)]
            m = jnp.maximum(jnp.maximum(logit[0], logit[1]),
                            jnp.maximum(logit[2], logit[3]))
            e = [jnp.exp(l - m) for l in logit]
            s = e[0] + e[1] + e[2] + e[3]
            o = (e[0] * vs[0] + e[1] * vs[1] + e[2] * vs[2] + e[3] * vs[3]) / s
            o_ref[:, h * DH:(h + 1) * DH] = o


def _dense(x, wts):
    nblk = N // BLK
    row_spec = pl.BlockSpec((BLK, D), lambda i: (i, 0))
    mat_spec = pl.BlockSpec((D, D), lambda i: (0, 0))
    vec_spec = pl.BlockSpec((1, D), lambda i: (0, 0))
    in_specs = [row_spec]
    for j in range(16):
        in_specs.append(mat_spec if j % 2 == 0 else vec_spec)
    return pl.pallas_call(
        _dense_body,
        grid=(nblk,),
        in_specs=in_specs,
        out_specs=[row_spec, row_spec],
        out_shape=[jax.ShapeDtypeStruct((N, D), jnp.float32)] * 2,
    )(x, *wts)


def _sc_aggregate_body(out_f_hbm, out_l_hbm, src_f_hbm, dst_f_hbm,
                       src_l_hbm, dst_l_hbm, zeros2_hbm, zeros1_hbm,
                       ones_hbm, sum_f_hbm, cnt_f_hbm, sum_l_hbm, cnt_l_hbm,
                       src_ring, dst_v, msg_a, msg_b, ones_v, acc_s, cnt_s,
                       sem_a, sem_b, sem_i0, sem_i1):
    cid = lax.axis_index("c")
    sid = lax.axis_index("s")
    row0 = sid * ROWS_PER_SUB
    pltpu.sync_copy(zeros2_hbm, acc_s.at[pl.ds(row0, ROWS_PER_SUB)])
    pltpu.sync_copy(zeros1_hbm, cnt_s.at[pl.ds(row0, ROWS_PER_SUB)])
    pltpu.sync_copy(ones_hbm, ones_v)
    plsc.subcore_barrier()

    msgs = (msg_a, msg_b)
    sems = (sem_a, sem_b)
    isems = (sem_i0, sem_i1)

    def run(table_hbm, src_hbm, dst_hbm):
        pltpu.sync_copy(dst_hbm.at[sid], dst_v)
        pltpu.sync_copy(src_hbm.at[sid, 0], src_ring.at[0])
        pltpu.async_copy(src_hbm.at[sid, 1], src_ring.at[1], sem_i1)
        pltpu.async_copy(table_hbm.at[src_ring.at[0]], msg_a, sem_a)

        def body(j, carry):
            cur = jnp.remainder(j, 2)

            def step(k):
                nxt = 1 - k
                pltpu.make_async_copy(
                    table_hbm.at[src_ring.at[jnp.remainder(j, 8)]],
                    msgs[k], sems[k]).wait()

                @pl.when(j + 1 < CHUNKS)
                def _():
                    s1 = jnp.remainder(j + 1, 8)
                    pltpu.make_async_copy(src_hbm.at[sid, j + 1],
                                          src_ring.at[s1], isems[nxt]).wait()

                    @pl.when(j + 2 < CHUNKS)
                    def _():
                        pltpu.async_copy(src_hbm.at[sid, j + 2],
                                         src_ring.at[jnp.remainder(j + 2, 8)],
                                         isems[k])

                    pltpu.async_copy(table_hbm.at[src_ring.at[s1]],
                                     msgs[nxt], sems[nxt])

                pltpu.sync_copy(msgs[k], acc_s.at[dst_v.at[j]], add=True)
                pltpu.sync_copy(ones_v, cnt_s.at[dst_v.at[j]], add=True)

            @pl.when(cur == 0)
            def _():
                step(0)

            @pl.when(cur == 1)
            def _():
                step(1)

            return carry

        lax.fori_loop(0, CHUNKS, body, 0)

    @pl.when(cid == 0)
    def _():
        run(out_f_hbm, src_f_hbm, dst_f_hbm)

    @pl.when(cid == 1)
    def _():
        run(out_l_hbm, src_l_hbm, dst_l_hbm)

    plsc.subcore_barrier()

    @pl.when(cid == 0)
    def _():
        pltpu.sync_copy(acc_s.at[pl.ds(row0, ROWS_PER_SUB)],
                        sum_f_hbm.at[pl.ds(row0, ROWS_PER_SUB)])
        pltpu.sync_copy(cnt_s.at[pl.ds(row0, ROWS_PER_SUB)],
                        cnt_f_hbm.at[pl.ds(row0, ROWS_PER_SUB)])

    @pl.when(cid == 1)
    def _():
        pltpu.sync_copy(acc_s.at[pl.ds(row0, ROWS_PER_SUB)],
                        sum_l_hbm.at[pl.ds(row0, ROWS_PER_SUB)])
        pltpu.sync_copy(cnt_s.at[pl.ds(row0, ROWS_PER_SUB)],
                        cnt_l_hbm.at[pl.ds(row0, ROWS_PER_SUB)])


@functools.lru_cache(maxsize=1)
def _make_sc_aggregate():
    mesh = plsc.VectorSubcoreMesh(core_axis_name="c", subcore_axis_name="s")
    return pl.kernel(
        _sc_aggregate_body,
        mesh=mesh,
        out_type=[
            jax.ShapeDtypeStruct((N_PAD, D), jnp.float32),
            jax.ShapeDtypeStruct((N_PAD,), jnp.float32),
            jax.ShapeDtypeStruct((N_PAD, D), jnp.float32),
            jax.ShapeDtypeStruct((N_PAD,), jnp.float32),
        ],
        scratch_types=[
            pltpu.VMEM((8, CHUNK), jnp.int32),
            pltpu.VMEM((CHUNKS, CHUNK), jnp.int32),
            pltpu.VMEM((CHUNK, D), jnp.float32),
            pltpu.VMEM((CHUNK, D), jnp.float32),
            pltpu.VMEM((CHUNK,), jnp.float32),
            pltpu.VMEM_SHARED((N_PAD, D), jnp.float32),
            pltpu.VMEM_SHARED((N_PAD,), jnp.float32),
            pltpu.SemaphoreType.DMA,
            pltpu.SemaphoreType.DMA,
            pltpu.SemaphoreType.DMA,
            pltpu.SemaphoreType.DMA,
        ],
    )


def _combine_body(sf_ref, cf_ref, sl_ref, cl_ref, x_ref, g_ref, b_ref, o_ref):
    h = (sf_ref[...] / jnp.maximum(cf_ref[...], 1.0)
         + sl_ref[...] / jnp.maximum(cl_ref[...], 1.0)
         + x_ref[...])
    mu = jnp.mean(h, axis=1, keepdims=True)
    d = h - mu
    var = jnp.mean(d * d, axis=1, keepdims=True)
    o_ref[...] = d * lax.rsqrt(var + 1e-5) * g_ref[...] + b_ref[...]


def _combine(sum_f, cnt_f, sum_l, cnt_l, x, gamma, beta):
    nblk = N // BLK
    row_spec = pl.BlockSpec((BLK, D), lambda i: (i, 0))
    col_spec = pl.BlockSpec((BLK, 1), lambda i: (i, 0))
    vec_spec = pl.BlockSpec((1, D), lambda i: (0, 0))
    return pl.pallas_call(
        _combine_body,
        grid=(nblk,),
        in_specs=[row_spec, col_spec, row_spec, col_spec, row_spec,
                  vec_spec, vec_spec],
        out_specs=row_spec,
        out_shape=jax.ShapeDtypeStruct((N, D), jnp.float32),
    )(sum_f, cnt_f, sum_l, cnt_l, x, gamma, beta)


def _prep_edges(ei):
    src = ei[0].astype(jnp.int32)
    dst = ei[1].astype(jnp.int32)
    pad = E_PAD - E
    src = jnp.concatenate([src, jnp.zeros((pad,), jnp.int32)])
    dst = jnp.concatenate([dst, jnp.full((pad,), N, jnp.int32)])
    return src.reshape(NS, CHUNKS, CHUNK), dst.reshape(NS, CHUNKS, CHUNK)


def kernel(x, edge_index_follows, edge_index_likes,
           W_follows, b_follows, Wq_follows, bq_follows, Wk_follows,
           bk_follows, Wv_follows, bv_follows,
           W_likes, b_likes, Wq_likes, bq_likes, Wk_likes, bk_likes,
           Wv_likes, bv_likes, ln_gamma, ln_beta):
    wts = []
    for (w, b, wq, bq, wk, bk, wv, bv) in (
            (W_follows, b_follows, Wq_follows, bq_follows, Wk_follows,
             bk_follows, Wv_follows, bv_follows),
            (W_likes, b_likes, Wq_likes, bq_likes, Wk_likes, bk_likes,
             Wv_likes, bv_likes)):
        wts += [w.T, b.reshape(1, D), wq.T, bq.reshape(1, D),
                wk.T, bk.reshape(1, D), wv.T, bv.reshape(1, D)]
    out_f, out_l = _dense(x, wts)

    src_f, dst_f = _prep_edges(edge_index_follows)
    src_l, dst_l = _prep_edges(edge_index_likes)
    zeros2 = jnp.zeros((ROWS_PER_SUB, D), jnp.float32)
    zeros1 = jnp.zeros((ROWS_PER_SUB,), jnp.float32)
    ones = jnp.ones((CHUNK,), jnp.float32)
    sum_f, cnt_f, sum_l, cnt_l = _make_sc_aggregate()(
        out_f, out_l, src_f, dst_f, src_l, dst_l, zeros2, zeros1, ones)

    return _combine(sum_f[:N], cnt_f[:N].reshape(N, 1),
                    sum_l[:N], cnt_l[:N].reshape(N, 1),
                    x, ln_gamma.reshape(1, D), ln_beta.reshape(1, D))


# speedup vs baseline: 5.1332x; 1.0037x over previous
import functools

import numpy as np
import jax
import jax.numpy as jnp
from jax import lax
from jax.experimental import pallas as pl
from jax.experimental.pallas import tpu as pltpu
from jax.experimental.pallas import tpu_sc as plsc

N = 10000
D = 128
E = 160000
H = 4
DH = D // H

NS = 16
CHUNK = 128
CHUNKS = 79
E_PAD = NS * CHUNKS * CHUNK
N_PAD = 10240
ROWS_PER_SUB = N_PAD // NS

BLK = 2000


def _dense_body(x_ref, *refs):
    wrefs = refs[:16]
    out_refs = refs[16:]
    x = x_ref[...]
    scale = 1.0 / np.sqrt(DH)
    for et in range(2):
        (w, b, wq, bq, wk, bk, wv, bv) = wrefs[8 * et:8 * et + 8]
        o_ref = out_refs[et]
        wh = jnp.dot(x, w[...], preferred_element_type=jnp.float32) + b[...]
        q = jnp.dot(wh, wq[...], preferred_element_type=jnp.float32) + bq[...]
        k = jnp.dot(wh, wk[...], preferred_element_type=jnp.float32) + bk[...]
        v = jnp.dot(wh, wv[...], preferred_element_type=jnp.float32) + bv[...]
        qs = [q[:, h * DH:(h + 1) * DH] for h in range(H)]
        ks = [k[:, h * DH:(h + 1) * DH] for h in range(H)]
        vs = [v[:, h * DH:(h + 1) * DH] for h in range(H)]
        for h in range(H):
            logit = [jnp.sum(qs[h] * ks[g], axis=1, keepdims=True) * scale
                     for g in range(H)]
            m = jnp.maximum(jnp.maximum(logit[0], logit[1]),
                            jnp.maximum(logit[2], logit[3]))
            e = [jnp.exp(l - m) for l in logit]
            s = e[0] + e[1] + e[2] + e[3]
            o = (e[0] * vs[0] + e[1] * vs[1] + e[2] * vs[2] + e[3] * vs[3]) / s
            o_ref[:, h * DH:(h + 1) * DH] = o


def _dense(x, wts):
    nblk = N // BLK
    row_spec = pl.BlockSpec((BLK, D), lambda i: (i, 0))
    mat_spec = pl.BlockSpec((D, D), lambda i: (0, 0))
    vec_spec = pl.BlockSpec((1, D), lambda i: (0, 0))
    in_specs = [row_spec]
    for j in range(16):
        in_specs.append(mat_spec if j % 2 == 0 else vec_spec)
    return pl.pallas_call(
        _dense_body,
        grid=(nblk,),
        in_specs=in_specs,
        out_specs=[row_spec, row_spec],
        out_shape=[jax.ShapeDtypeStruct((N, D), jnp.float32)] * 2,
    )(x, *wts)


def _sc_aggregate_body(out_f_hbm, out_l_hbm, src_f_hbm, dst_f_hbm,
                       src_l_hbm, dst_l_hbm, zeros2_hbm, zeros1_hbm,
                       ones_hbm, sum_f_hbm, cnt_f_hbm, sum_l_hbm, cnt_l_hbm,
                       src_ring, dst_v, msg_a, msg_b, ones_v, acc_s, cnt_s,
                       sem_a, sem_b, sem_i0, sem_i1, sem_s, sem_c):
    cid = lax.axis_index("c")
    sid = lax.axis_index("s")
    row0 = sid * ROWS_PER_SUB
    pltpu.sync_copy(zeros2_hbm, acc_s.at[pl.ds(row0, ROWS_PER_SUB)])
    pltpu.sync_copy(zeros1_hbm, cnt_s.at[pl.ds(row0, ROWS_PER_SUB)])
    pltpu.sync_copy(ones_hbm, ones_v)
    plsc.subcore_barrier()

    msgs = (msg_a, msg_b)
    sems = (sem_a, sem_b)
    isems = (sem_i0, sem_i1)

    def run(table_hbm, src_hbm, dst_hbm):
        pltpu.sync_copy(dst_hbm.at[sid], dst_v)
        pltpu.sync_copy(src_hbm.at[sid, 0], src_ring.at[0])
        pltpu.async_copy(src_hbm.at[sid, 1], src_ring.at[1], sem_i1)
        pltpu.async_copy(table_hbm.at[src_ring.at[0]], msg_a, sem_a)

        def body(j, carry):
            cur = jnp.remainder(j, 2)

            def step(k):
                nxt = 1 - k
                pltpu.make_async_copy(
                    table_hbm.at[src_ring.at[jnp.remainder(j, 8)]],
                    msgs[k], sems[k]).wait()

                @pl.when(j + 1 < CHUNKS)
                def _():
                    s1 = jnp.remainder(j + 1, 8)
                    pltpu.make_async_copy(src_hbm.at[sid, j + 1],
                                          src_ring.at[s1], isems[nxt]).wait()

                    @pl.when(j + 2 < CHUNKS)
                    def _():
                        pltpu.async_copy(src_hbm.at[sid, j + 2],
                                         src_ring.at[jnp.remainder(j + 2, 8)],
                                         isems[k])

                    @pl.when(j >= 1)
                    def _():
                        pltpu.make_async_copy(
                            msgs[nxt], acc_s.at[dst_v.at[j]], sem_s).wait()

                    pltpu.async_copy(table_hbm.at[src_ring.at[s1]],
                                     msgs[nxt], sems[nxt])

                @pl.when(j >= 1)
                def _():
                    pltpu.make_async_copy(ones_v, cnt_s.at[dst_v.at[j]],
                                          sem_c).wait()

                pltpu.async_copy(msgs[k], acc_s.at[dst_v.at[j]], sem_s,
                                 add=True)
                pltpu.async_copy(ones_v, cnt_s.at[dst_v.at[j]], sem_c,
                                 add=True)

            @pl.when(cur == 0)
            def _():
                step(0)

            @pl.when(cur == 1)
            def _():
                step(1)

            return carry

        lax.fori_loop(0, CHUNKS, body, 0)
        pltpu.make_async_copy(msg_a, acc_s.at[dst_v.at[0]], sem_s).wait()
        pltpu.make_async_copy(msg_a, acc_s.at[dst_v.at[0]], sem_s).wait()
        pltpu.make_async_copy(ones_v, cnt_s.at[dst_v.at[0]], sem_c).wait()

    @pl.when(cid == 0)
    def _():
        run(out_f_hbm, src_f_hbm, dst_f_hbm)

    @pl.when(cid == 1)
    def _():
        run(out_l_hbm, src_l_hbm, dst_l_hbm)

    plsc.subcore_barrier()

    @pl.when(cid == 0)
    def _():
        pltpu.sync_copy(acc_s.at[pl.ds(row0, ROWS_PER_SUB)],
                        sum_f_hbm.at[pl.ds(row0, ROWS_PER_SUB)])
        pltpu.sync_copy(cnt_s.at[pl.ds(row0, ROWS_PER_SUB)],
                        cnt_f_hbm.at[pl.ds(row0, ROWS_PER_SUB)])

    @pl.when(cid == 1)
    def _():
        pltpu.sync_copy(acc_s.at[pl.ds(row0, ROWS_PER_SUB)],
                        sum_l_hbm.at[pl.ds(row0, ROWS_PER_SUB)])
        pltpu.sync_copy(cnt_s.at[pl.ds(row0, ROWS_PER_SUB)],
                        cnt_l_hbm.at[pl.ds(row0, ROWS_PER_SUB)])


@functools.lru_cache(maxsize=1)
def _make_sc_aggregate():
    mesh = plsc.VectorSubcoreMesh(core_axis_name="c", subcore_axis_name="s")
    return pl.kernel(
        _sc_aggregate_body,
        mesh=mesh,
        out_type=[
            jax.ShapeDtypeStruct((N_PAD, D), jnp.float32),
            jax.ShapeDtypeStruct((N_PAD,), jnp.float32),
            jax.ShapeDtypeStruct((N_PAD, D), jnp.float32),
            jax.ShapeDtypeStruct((N_PAD,), jnp.float32),
        ],
        scratch_types=[
            pltpu.VMEM((8, CHUNK), jnp.int32),
            pltpu.VMEM((CHUNKS, CHUNK), jnp.int32),
            pltpu.VMEM((CHUNK, D), jnp.float32),
            pltpu.VMEM((CHUNK, D), jnp.float32),
            pltpu.VMEM((CHUNK,), jnp.float32),
            pltpu.VMEM_SHARED((N_PAD, D), jnp.float32),
            pltpu.VMEM_SHARED((N_PAD,), jnp.float32),
            pltpu.SemaphoreType.DMA,
            pltpu.SemaphoreType.DMA,
            pltpu.SemaphoreType.DMA,
            pltpu.SemaphoreType.DMA,
            pltpu.SemaphoreType.DMA,
            pltpu.SemaphoreType.DMA,
        ],
    )


def _combine_body(sf_ref, cf_ref, sl_ref, cl_ref, x_ref, g_ref, b_ref, o_ref):
    h = (sf_ref[...] / jnp.maximum(cf_ref[...], 1.0)
         + sl_ref[...] / jnp.maximum(cl_ref[...], 1.0)
         + x_ref[...])
    mu = jnp.mean(h, axis=1, keepdims=True)
    d = h - mu
    var = jnp.mean(d * d, axis=1, keepdims=True)
    o_ref[...] = d * lax.rsqrt(var + 1e-5) * g_ref[...] + b_ref[...]


def _combine(sum_f, cnt_f, sum_l, cnt_l, x, gamma, beta):
    nblk = N // BLK
    row_spec = pl.BlockSpec((BLK, D), lambda i: (i, 0))
    col_spec = pl.BlockSpec((BLK, 1), lambda i: (i, 0))
    vec_spec = pl.BlockSpec((1, D), lambda i: (0, 0))
    return pl.pallas_call(
        _combine_body,
        grid=(nblk,),
        in_specs=[row_spec, col_spec, row_spec, col_spec, row_spec,
                  vec_spec, vec_spec],
        out_specs=row_spec,
        out_shape=jax.ShapeDtypeStruct((N, D), jnp.float32),
    )(sum_f, cnt_f, sum_l, cnt_l, x, gamma, beta)


def _prep_edges(ei):
    src = ei[0].astype(jnp.int32)
    dst = ei[1].astype(jnp.int32)
    pad = E_PAD - E
    src = jnp.concatenate([src, jnp.zeros((pad,), jnp.int32)])
    dst = jnp.concatenate([dst, jnp.full((pad,), N, jnp.int32)])
    return src.reshape(NS, CHUNKS, CHUNK), dst.reshape(NS, CHUNKS, CHUNK)


def kernel(x, edge_index_follows, edge_index_likes,
           W_follows, b_follows, Wq_follows, bq_follows, Wk_follows,
           bk_follows, Wv_follows, bv_follows,
           W_likes, b_likes, Wq_likes, bq_likes, Wk_likes, bk_likes,
           Wv_likes, bv_likes, ln_gamma, ln_beta):
    wts = []
    for (w, b, wq, bq, wk, bk, wv, bv) in (
            (W_follows, b_follows, Wq_follows, bq_follows, Wk_follows,
             bk_follows, Wv_follows, bv_follows),
            (W_likes, b_likes, Wq_likes, bq_likes, Wk_likes, bk_likes,
             Wv_likes, bv_likes)):
        wts += [w.T, b.reshape(1, D), wq.T, bq.reshape(1, D),
                wk.T, bk.reshape(1, D), wv.T, bv.reshape(1, D)]
    out_f, out_l = _dense(x, wts)

    src_f, dst_f = _prep_edges(edge_index_follows)
    src_l, dst_l = _prep_edges(edge_index_likes)
    zeros2 = jnp.zeros((ROWS_PER_SUB, D), jnp.float32)
    zeros1 = jnp.zeros((ROWS_PER_SUB,), jnp.float32)
    ones = jnp.ones((CHUNK,), jnp.float32)
    sum_f, cnt_f, sum_l, cnt_l = _make_sc_aggregate()(
        out_f, out_l, src_f, dst_f, src_l, dst_l, zeros2, zeros1, ones)

    return _combine(sum_f[:N], cnt_f[:N].reshape(N, 1),
                    sum_l[:N], cnt_l[:N].reshape(N, 1),
                    x, ln_gamma.reshape(1, D), ln_beta.reshape(1, D))
